# baseline (speedup 1.0000x reference)
import functools

import jax
import jax.numpy as jnp
from jax import lax
from jax.experimental import pallas as pl
from jax.experimental.pallas import tpu as pltpu
from jax.experimental.pallas import tpu_sc as plsc

_DIM = 64
_GR = 40
_NBUF = 4


@functools.cache
def _make_kernel(batch, hist):
    info = plsc.get_sparse_core_info()
    nc, ns = info.num_cores, info.num_subcores
    nw = nc * ns
    epw = batch // nw
    bpw = epw * hist
    gpc = hist // _GR
    nouter = epw // _NBUF
    mesh = plsc.VectorSubcoreMesh(core_axis_name="c", subcore_axis_name="s")

    scratch = [pltpu.VMEM((bpw,), jnp.int32)]
    scratch += [pltpu.VMEM((hist, _DIM), jnp.float32) for _ in range(_NBUF)]
    scratch += [pltpu.SemaphoreType.DMA for _ in range(2 * _NBUF + 1)]

    @functools.partial(
        pl.kernel,
        mesh=mesh,
        out_type=jax.ShapeDtypeStruct((batch, hist, _DIM), jnp.float32),
        scratch_types=scratch,
        compiler_params=pltpu.CompilerParams(use_tc_tiling_on_sc=False),
    )
    def k(idx_hbm, table_hbm, out_hbm, idx_v, *bufs_and_sems):
        gbuf = bufs_and_sems[:_NBUF]
        gsem = bufs_and_sems[_NBUF:2 * _NBUF]
        wsem = bufs_and_sems[2 * _NBUF:3 * _NBUF]
        isem = bufs_and_sems[3 * _NBUF]

        wid = lax.axis_index("s") * nc + lax.axis_index("c")
        base_row = wid * bpw
        base_el = wid * epw

        pltpu.async_copy(
            idx_hbm.at[pl.ds(base_row, bpw)], idx_v, isem
        ).wait()

        def fire_gathers(g, b):
            for j in range(gpc):
                pltpu.async_copy(
                    table_hbm.at[idx_v.at[pl.ds(g * hist + j * _GR, _GR)]],
                    gbuf[b].at[pl.ds(j * _GR, _GR)],
                    gsem[b],
                )

        def drain_gathers(b):
            pltpu.make_async_copy(
                table_hbm.at[pl.ds(0, hist)], gbuf[b], gsem[b]
            ).wait()

        def fire_writeback(g, b):
            el = base_el + g
            pltpu.async_copy(gbuf[b], out_hbm.at[el], wsem[b])

        def wait_writeback(b):
            pltpu.make_async_copy(gbuf[b], out_hbm.at[0], wsem[b]).wait()

        for b in range(_NBUF):
            fire_gathers(b, b)

        def body(s, carry):
            for b in range(_NBUF):
                g = s * _NBUF + b
                drain_gathers(b)
                fire_writeback(g, b)
                wait_writeback(b)
                fire_gathers(g + _NBUF, b)
            return carry

        lax.fori_loop(0, nouter - 1, body, 0)

        for b in range(_NBUF):
            g = (nouter - 1) * _NBUF + b
            drain_gathers(b)
            fire_writeback(g, b)
        for b in range(_NBUF):
            wait_writeback(b)

    return k


def kernel(to_embed, table):
    batch, hist = to_embed.shape
    idx = to_embed.reshape(-1).astype(jnp.int32)
    return _make_kernel(batch, hist)(idx, table)


# speedup vs baseline: 1.2234x; 1.2234x over previous
import functools

import jax
import jax.numpy as jnp
from jax import lax
from jax.experimental import pallas as pl
from jax.experimental.pallas import tpu as pltpu
from jax.experimental.pallas import tpu_sc as plsc

_DIM = 64
_CH = 128
_NBUF = 4


@functools.cache
def _make_kernel(B):
    info = plsc.get_sparse_core_info()
    nc, ns = info.num_cores, info.num_subcores
    nw = nc * ns
    bpw = B // nw
    nidx = bpw // _CH
    nchunk = bpw // _CH
    nouter = nchunk // _NBUF
    mesh = plsc.VectorSubcoreMesh(core_axis_name="c", subcore_axis_name="s")

    scratch = [pltpu.VMEM((nidx, _CH), jnp.int32)]
    scratch += [pltpu.VMEM((_CH, 128), jnp.float32) for _ in range(_NBUF)]
    scratch += [pltpu.SemaphoreType.DMA for _ in range(2 * _NBUF + 1)]

    @functools.partial(
        pl.kernel,
        mesh=mesh,
        out_type=jax.ShapeDtypeStruct((B, 128), jnp.float32),
        scratch_types=scratch,
        compiler_params=pltpu.CompilerParams(use_tc_tiling_on_sc=False),
    )
    def k(idx_hbm, table_hbm, out_hbm, idx_v, *bufs_and_sems):
        rows = bufs_and_sems[:_NBUF]
        gsem = bufs_and_sems[_NBUF:2 * _NBUF]
        wsem = bufs_and_sems[2 * _NBUF:3 * _NBUF]
        isem = bufs_and_sems[3 * _NBUF]

        wid = lax.axis_index("s") * nc + lax.axis_index("c")
        base = wid * bpw

        pltpu.async_copy(
            idx_hbm.at[pl.ds(wid * nidx, nidx)], idx_v, isem
        ).wait()

        def fire_gather(g, b):
            pltpu.async_copy(table_hbm.at[idx_v.at[g]], rows[b], gsem[b])

        def drain_gather(b):
            pltpu.make_async_copy(
                table_hbm.at[pl.ds(0, _CH)], rows[b], gsem[b]
            ).wait()

        def fire_writeback(g, b):
            off = pl.multiple_of(base + g * _CH, _CH)
            pltpu.async_copy(rows[b], out_hbm.at[pl.ds(off, _CH)], wsem[b])

        def wait_writeback(b):
            pltpu.make_async_copy(
                rows[b], out_hbm.at[pl.ds(0, _CH)], wsem[b]
            ).wait()

        for b in range(_NBUF):
            fire_gather(b, b)

        def body(s, carry):
            for b in range(_NBUF):
                g = s * _NBUF + b
                drain_gather(b)
                fire_writeback(g, b)
                wait_writeback(b)
                fire_gather(g + _NBUF, b)
            return carry

        lax.fori_loop(0, nouter - 1, body, 0)

        for b in range(_NBUF):
            g = (nouter - 1) * _NBUF + b
            drain_gather(b)
            fire_writeback(g, b)
        for b in range(_NBUF):
            wait_writeback(b)

    return k


def kernel(to_embed, table):
    batch, hist = to_embed.shape
    b = batch * hist
    idx = to_embed.reshape(b // _CH, _CH).astype(jnp.int32)
    table_pad = jnp.pad(table, ((0, 0), (0, 128 - _DIM)))
    out = _make_kernel(b)(idx, table_pad)
    return out.reshape(batch, hist, 128)[:, :, :_DIM]


# speedup vs baseline: 1.3154x; 1.0752x over previous
import functools

import jax
import jax.numpy as jnp
from jax import lax
from jax.experimental import pallas as pl
from jax.experimental.pallas import tpu as pltpu
from jax.experimental.pallas import tpu_sc as plsc

_DIM = 64
_CH = 128
_NBUF = 5


@functools.cache
def _make_kernel(B):
    info = plsc.get_sparse_core_info()
    nc, ns = info.num_cores, info.num_subcores
    nw = nc * ns
    bpw = B // nw
    nidx = bpw // _CH
    nchunk = bpw // _CH
    nouter = nchunk // _NBUF
    mesh = plsc.VectorSubcoreMesh(core_axis_name="c", subcore_axis_name="s")

    scratch = [pltpu.VMEM((nidx, _CH), jnp.int32)]
    scratch += [pltpu.VMEM((_CH, 128), jnp.float32) for _ in range(_NBUF)]
    scratch += [pltpu.SemaphoreType.DMA for _ in range(2 * _NBUF + 1)]

    @functools.partial(
        pl.kernel,
        mesh=mesh,
        out_type=jax.ShapeDtypeStruct((B, 128), jnp.float32),
        scratch_types=scratch,
        compiler_params=pltpu.CompilerParams(use_tc_tiling_on_sc=False),
    )
    def k(idx_hbm, table_hbm, out_hbm, idx_v, *bufs_and_sems):
        rows = bufs_and_sems[:_NBUF]
        gsem = bufs_and_sems[_NBUF:2 * _NBUF]
        wsem = bufs_and_sems[2 * _NBUF:3 * _NBUF]
        isem = bufs_and_sems[3 * _NBUF]

        wid = lax.axis_index("s") * nc + lax.axis_index("c")
        base = wid * bpw

        pltpu.async_copy(
            idx_hbm.at[pl.ds(wid * nidx, nidx)], idx_v, isem
        ).wait()

        def fire_gather(g, b):
            pltpu.async_copy(table_hbm.at[idx_v.at[g]], rows[b], gsem[b])

        def drain_gather(b):
            pltpu.make_async_copy(
                table_hbm.at[pl.ds(0, _CH)], rows[b], gsem[b]
            ).wait()

        def fire_writeback(g, b):
            off = pl.multiple_of(base + g * _CH, _CH)
            pltpu.async_copy(
                rows[b].at[:, pl.ds(0, _DIM)],
                out_hbm.at[pl.ds(off, _CH), pl.ds(0, _DIM)],
                wsem[b],
            )

        def wait_writeback(b):
            pltpu.make_async_copy(
                rows[b].at[:, pl.ds(0, _DIM)],
                out_hbm.at[pl.ds(0, _CH), pl.ds(0, _DIM)],
                wsem[b],
            ).wait()

        for b in range(_NBUF):
            fire_gather(b, b)

        def body(s, carry):
            for b in range(_NBUF):
                g = s * _NBUF + b
                drain_gather(b)
                fire_writeback(g, b)
                wait_writeback(b)
                fire_gather(g + _NBUF, b)
            return carry

        lax.fori_loop(0, nouter - 1, body, 0)

        for b in range(_NBUF):
            g = (nouter - 1) * _NBUF + b
            drain_gather(b)
            fire_writeback(g, b)
        for b in range(_NBUF):
            wait_writeback(b)

    return k


def kernel(to_embed, table):
    batch, hist = to_embed.shape
    b = batch * hist
    idx = to_embed.reshape(b // _CH, _CH).astype(jnp.int32)
    table_pad = jnp.pad(table, ((0, 0), (0, 128 - _DIM)))
    out = _make_kernel(b)(idx, table_pad)
    return out.reshape(batch, hist, 128)[:, :, :_DIM]
